# baseline (speedup 1.0000x reference)
import functools

import jax
import jax.numpy as jnp
from jax import lax
from jax.experimental import pallas as pl
from jax.experimental.pallas import tpu as pltpu
from jax.experimental.pallas import tpu_sc as plsc

N_NODES = 10000
N_EDGES = 160000
F_IN = 128
F_EDGE = 16
HID = 16
N_CORES = 2
N_SUB = 16
NW = N_CORES * N_SUB
CHUNK = 128
NCHUNK = 40
EPW = NCHUNK * CHUNK
E_PAD = NW * EPW

@functools.lru_cache(maxsize=None)
def _mesh():
    return plsc.VectorSubcoreMesh(
        core_axis_name="c", subcore_axis_name="s",
        num_cores=N_CORES, num_subcores=N_SUB,
    )


NCHUNK_FAST = 64
NCHUNK_SLOW = 16
N_CHUNKS = E_PAD // CHUNK


@functools.lru_cache(maxsize=None)
def _make_gather(d, fast_core):
    @functools.partial(
        pl.kernel,
        out_type=jax.ShapeDtypeStruct((E_PAD, d), jnp.float32),
        mesh=_mesh(),
        scratch_types=[
            pltpu.VMEM((NCHUNK_FAST, CHUNK), jnp.int32),
            pltpu.VMEM((CHUNK, d), jnp.float32),
            pltpu.VMEM((CHUNK, d), jnp.float32),
            pltpu.SemaphoreType.DMA,
            pltpu.SemaphoreType.DMA,
        ],
    )
    def gather_k(table, idx, out, idx_v, rows_a, rows_b, sg_a, sg_b):
        cid = lax.axis_index("c")
        sid = lax.axis_index("s")
        is_fast = cid == fast_core
        my_n = jnp.where(is_fast, NCHUNK_FAST, NCHUNK_SLOW)
        base_chunk = jnp.where(
            is_fast, sid * NCHUNK_FAST,
            N_SUB * NCHUNK_FAST + sid * NCHUNK_SLOW,
        )

        @pl.when(is_fast)
        def _():
            pltpu.sync_copy(idx.at[pl.ds(sid * NCHUNK_FAST, NCHUNK_FAST)], idx_v)

        @pl.when(jnp.logical_not(is_fast))
        def _():
            pltpu.sync_copy(
                idx.at[pl.ds(N_SUB * NCHUNK_FAST + sid * NCHUNK_SLOW, NCHUNK_SLOW)],
                idx_v.at[pl.ds(0, NCHUNK_SLOW)],
            )

        rows = (rows_a, rows_b)
        sg = (sg_a, sg_b)
        pltpu.async_copy(table.at[idx_v.at[0]], rows_a, sg_a)

        def step(g, carry):
            @pl.when(g < my_n)
            def _():
                cur = lax.rem(g, 2)

                @pl.when(g + 1 < my_n)
                def _():
                    for bsel in range(2):
                        @pl.when(cur == bsel)
                        def _():
                            pltpu.async_copy(
                                table.at[idx_v.at[g + 1]], rows[1 - bsel],
                                sg[1 - bsel],
                            )

                for bsel in range(2):
                    @pl.when(cur == bsel)
                    def _():
                        pltpu.make_async_copy(
                            table.at[idx_v.at[g]], rows[bsel], sg[bsel]
                        ).wait()
                        pltpu.sync_copy(
                            rows[bsel],
                            out.at[pl.ds((base_chunk + g) * CHUNK, CHUNK)],
                        )
            return carry

        lax.fori_loop(0, NCHUNK_FAST, step, 0)

    return gather_k


@functools.lru_cache(maxsize=None)
def _make_scatter(w):
    @functools.partial(
        pl.kernel,
        out_type=jax.ShapeDtypeStruct((N_CORES, N_NODES, w), jnp.float32),
        mesh=_mesh(),
        scratch_types=[
            pltpu.VMEM((NCHUNK, CHUNK), jnp.int32),
            pltpu.VMEM((CHUNK,), jnp.int32),
            pltpu.VMEM((CHUNK, w), jnp.float32),
            pltpu.VMEM_SHARED((N_NODES, w), jnp.float32),
            pltpu.SemaphoreType.DMA,
        ],
    )
    def scatter_k(msg, idx, zeros, out, idx_v, idx_cur, rows_v, acc_sh, sem):
        cid = lax.axis_index("c")
        sid = lax.axis_index("s")
        wid = cid * N_SUB + sid

        @pl.when(sid == 0)
        def _():
            pltpu.sync_copy(zeros, acc_sh)

        plsc.subcore_barrier()
        pltpu.sync_copy(idx.at[wid], idx_v)
        base = wid * EPW

        def step(i, carry):
            pltpu.sync_copy(msg.at[pl.ds(base + i * CHUNK, CHUNK)], rows_v)
            for j in range(CHUNK // 16):
                idx_cur[pl.ds(j * 16, 16)] = idx_v[i, pl.ds(j * 16, 16)]
            pltpu.sync_copy(rows_v, acc_sh.at[idx_cur], add=True)
            return carry

        lax.fori_loop(0, NCHUNK, step, 0)
        plsc.subcore_barrier()

        @pl.when(sid == 0)
        def _():
            pltpu.sync_copy(acc_sh, out.at[cid])

    return scatter_k


_EDGE_BLK = 1280
_MSG_W = 128


def _edge_body(ea_ref, xs_ref, w1_ref, b1_ref, wext_ref, rh_ref, f_ref,
               out_ref, *, with_ones):
    pid = pl.program_id(0)
    b = ea_ref.shape[0]
    row = pid * b + lax.broadcasted_iota(jnp.int32, (b, 1), 0)
    valid = (row < N_EDGES).astype(jnp.float32)
    h = jnp.maximum(
        jnp.dot(ea_ref[...], w1_ref[...], preferred_element_type=jnp.float32)
        + b1_ref[...],
        0.0,
    )
    hext = jnp.concatenate(
        [h, jnp.ones((b, 1), jnp.float32), jnp.zeros((b, 6), jnp.float32)], axis=1
    )
    y = jnp.dot(xs_ref[...], wext_ref[...], preferred_element_type=jnp.float32)
    hrep = jnp.dot(hext, rh_ref[...], preferred_element_type=jnp.float32)
    t = y * hrep
    t4 = t[:, 0:128] + t[:, 128:256] + t[:, 256:384] + t[:, 384:512]
    msg = jnp.dot(t4, f_ref[...], preferred_element_type=jnp.float32) * valid
    if with_ones:
        col = lax.broadcasted_iota(jnp.int32, (1, _MSG_W), 1)
        msg = msg + valid * (col == 16).astype(jnp.float32)
    out_ref[...] = msg


def _make_edge(c_in, with_ones):
    grid = E_PAD // _EDGE_BLK
    return pl.pallas_call(
        functools.partial(_edge_body, with_ones=with_ones),
        grid=(grid,),
        in_specs=[
            pl.BlockSpec((_EDGE_BLK, F_EDGE),
                         lambda i: (jnp.minimum(i, N_EDGES // _EDGE_BLK - 1), 0)),
            pl.BlockSpec((_EDGE_BLK, c_in), lambda i: (i, 0)),
            pl.BlockSpec((F_EDGE, 25), lambda i: (0, 0)),
            pl.BlockSpec((1, 25), lambda i: (0, 0)),
            pl.BlockSpec((c_in, 512), lambda i: (0, 0)),
            pl.BlockSpec((32, 512), lambda i: (0, 0)),
            pl.BlockSpec((128, _MSG_W), lambda i: (0, 0)),
        ],
        out_specs=pl.BlockSpec((_EDGE_BLK, _MSG_W), lambda i: (i, 0)),
        out_shape=jax.ShapeDtypeStruct((E_PAD, _MSG_W), jnp.float32),
    )


_edge1 = _make_edge(F_IN, True)
_edge23 = _make_edge(F_IN, False)


def _node1_body(p_ref, x_ref, root_ref, bias_ref, h_ref, invc_ref):
    s = p_ref[0] + p_ref[1]
    invc = 1.0 / jnp.maximum(s[:, 16:17], 1.0)
    val = (
        s[:, :16] * invc
        + jnp.dot(x_ref[...], root_ref[...], preferred_element_type=jnp.float32)
        + bias_ref[...]
    )
    h_ref[...] = jnp.concatenate(
        [jnp.maximum(val, 0.0), jnp.zeros((N_NODES, F_IN - HID), jnp.float32)],
        axis=1,
    )
    invc_ref[...] = invc


_node1 = pl.pallas_call(
    _node1_body,
    out_shape=(
        jax.ShapeDtypeStruct((N_NODES, F_IN), jnp.float32),
        jax.ShapeDtypeStruct((N_NODES, 1), jnp.float32),
    ),
)


def _node23_body(p_ref, act_ref, invc_ref, root_ref, bias_ref, out_ref, *, relu):
    s = (p_ref[0] + p_ref[1])[:, :HID]
    val = (
        s * invc_ref[...]
        + jnp.dot(act_ref[...], root_ref[...], preferred_element_type=jnp.float32)
        + bias_ref[...]
    )
    if relu:
        out_ref[...] = jnp.concatenate(
            [jnp.maximum(val, 0.0), jnp.zeros((N_NODES, F_IN - HID), jnp.float32)],
            axis=1,
        )
    else:
        out_ref[...] = val


def _make_node23(relu):
    return pl.pallas_call(
        functools.partial(_node23_body, relu=relu),
        out_shape=jax.ShapeDtypeStruct(
            (N_NODES, F_IN if relu else HID), jnp.float32
        ),
    )


_node2 = _make_node23(True)
_node3 = _make_node23(False)


def _pack_wext(w2, b2, c_in):
    w = w2.reshape(25, c_in, 16).transpose(1, 0, 2).reshape(c_in, 400)
    w = jnp.concatenate(
        [w, b2.reshape(c_in, 16), jnp.zeros((c_in, 96), jnp.float32)], axis=1
    )
    if c_in < F_IN:
        w = jnp.concatenate([w, jnp.zeros((F_IN - c_in, 512), jnp.float32)], 0)
    return w


def _selectors():
    k = jnp.arange(32)[:, None]
    j = jnp.arange(512)[None, :]
    rh = (j // 16 == k).astype(jnp.float32)
    jj = jnp.arange(128)[:, None]
    oo = jnp.arange(_MSG_W)[None, :]
    f = ((jj % 16 == oo) & (oo < 16)).astype(jnp.float32)
    return rh, f


def _pad_root(root, c_in):
    if c_in < F_IN:
        root = jnp.concatenate(
            [root, jnp.zeros((F_IN - c_in, HID), jnp.float32)], 0
        )
    return root


def kernel(x, edge_index, edge_attr,
           c1_w1, c1_b1, c1_w2, c1_b2, c1_root, c1_bias,
           c2_w1, c2_b1, c2_w2, c2_b2, c2_root, c2_bias,
           c3_w1, c3_b1, c3_w2, c3_b2, c3_root, c3_bias):
    pad = jnp.zeros((2, E_PAD - N_EDGES), jnp.int32)
    ei = jnp.concatenate([edge_index, pad], axis=1)
    src = ei[0].reshape(E_PAD // CHUNK, CHUNK)
    dst = ei[1].reshape(NW, NCHUNK, CHUNK)
    ea = edge_attr
    zeros = jnp.zeros((N_NODES, _MSG_W), jnp.float32)

    wext1 = _pack_wext(c1_w2, c1_b2, F_IN)
    wext2 = _pack_wext(c2_w2, c2_b2, HID)
    wext3 = _pack_wext(c3_w2, c3_b2, HID)
    rh, f = _selectors()

    xsrc = _make_gather(F_IN, 0)(x, src)
    msg1 = _edge1(ea, xsrc, c1_w1, c1_b1.reshape(1, 25), wext1, rh, f)
    p1 = _make_scatter(_MSG_W)(msg1, dst, zeros)
    h1, invc = _node1(p1, x, c1_root, c1_bias.reshape(1, HID))

    hs1 = _make_gather(F_IN, 0)(h1, src)
    msg2 = _edge23(ea, hs1, c2_w1, c2_b1.reshape(1, 25), wext2, rh, f)
    p2 = _make_scatter(_MSG_W)(msg2, dst, zeros)
    h2 = _node2(p2, h1, invc, _pad_root(c2_root, HID), c2_bias.reshape(1, HID))

    hs2 = _make_gather(F_IN, 0)(h2, src)
    msg3 = _edge23(ea, hs2, c3_w1, c3_b1.reshape(1, 25), wext3, rh, f)
    p3 = _make_scatter(_MSG_W)(msg3, dst, zeros)
    out = _node3(p3, h2, invc, _pad_root(c3_root, HID), c3_bias.reshape(1, HID))
    return out


# speedup vs baseline: 1.0239x; 1.0239x over previous
import functools

import jax
import jax.numpy as jnp
from jax import lax
from jax.experimental import pallas as pl
from jax.experimental.pallas import tpu as pltpu
from jax.experimental.pallas import tpu_sc as plsc

N_NODES = 10000
N_EDGES = 160000
F_IN = 128
F_EDGE = 16
HID = 16
N_CORES = 2
N_SUB = 16
NW = N_CORES * N_SUB
CHUNK = 128
NCHUNK = 40
EPW = NCHUNK * CHUNK
E_PAD = NW * EPW

@functools.lru_cache(maxsize=None)
def _mesh():
    return plsc.VectorSubcoreMesh(
        core_axis_name="c", subcore_axis_name="s",
        num_cores=N_CORES, num_subcores=N_SUB,
    )


NCHUNK_FAST = 64
NCHUNK_SLOW = 16
N_CHUNKS = E_PAD // CHUNK


@functools.lru_cache(maxsize=None)
def _make_gather(d, fast_core):
    @functools.partial(
        pl.kernel,
        out_type=jax.ShapeDtypeStruct((E_PAD, d), jnp.float32),
        mesh=_mesh(),
        scratch_types=[
            pltpu.VMEM((NCHUNK_FAST, CHUNK), jnp.int32),
            pltpu.VMEM((CHUNK, d), jnp.float32),
            pltpu.VMEM((CHUNK, d), jnp.float32),
            pltpu.SemaphoreType.DMA,
            pltpu.SemaphoreType.DMA,
        ],
    )
    def gather_k(table, idx, out, idx_v, rows_a, rows_b, sg_a, sg_b):
        cid = lax.axis_index("c")
        sid = lax.axis_index("s")
        is_fast = cid == fast_core
        my_n = jnp.where(is_fast, NCHUNK_FAST, NCHUNK_SLOW)
        base_chunk = jnp.where(
            is_fast, sid * NCHUNK_FAST,
            N_SUB * NCHUNK_FAST + sid * NCHUNK_SLOW,
        )

        @pl.when(is_fast)
        def _():
            pltpu.sync_copy(idx.at[pl.ds(sid * NCHUNK_FAST, NCHUNK_FAST)], idx_v)

        @pl.when(jnp.logical_not(is_fast))
        def _():
            pltpu.sync_copy(
                idx.at[pl.ds(N_SUB * NCHUNK_FAST + sid * NCHUNK_SLOW, NCHUNK_SLOW)],
                idx_v.at[pl.ds(0, NCHUNK_SLOW)],
            )

        rows = (rows_a, rows_b)
        sg = (sg_a, sg_b)
        pltpu.async_copy(table.at[idx_v.at[0]], rows_a, sg_a)

        def step(g, carry):
            @pl.when(g < my_n)
            def _():
                cur = lax.rem(g, 2)

                @pl.when(g + 1 < my_n)
                def _():
                    for bsel in range(2):
                        @pl.when(cur == bsel)
                        def _():
                            pltpu.async_copy(
                                table.at[idx_v.at[g + 1]], rows[1 - bsel],
                                sg[1 - bsel],
                            )

                for bsel in range(2):
                    @pl.when(cur == bsel)
                    def _():
                        pltpu.make_async_copy(
                            table.at[idx_v.at[g]], rows[bsel], sg[bsel]
                        ).wait()
                        pltpu.sync_copy(
                            rows[bsel],
                            out.at[pl.ds((base_chunk + g) * CHUNK, CHUNK)],
                        )
            return carry

        lax.fori_loop(0, NCHUNK_FAST, step, 0)

    return gather_k


@functools.lru_cache(maxsize=None)
def _make_scatter(w):
    @functools.partial(
        pl.kernel,
        out_type=jax.ShapeDtypeStruct((N_CORES, N_NODES, w), jnp.float32),
        mesh=_mesh(),
        scratch_types=[
            pltpu.VMEM((NCHUNK, CHUNK), jnp.int32),
            pltpu.VMEM((CHUNK,), jnp.int32),
            pltpu.VMEM((CHUNK, w), jnp.float32),
            pltpu.VMEM_SHARED((N_NODES, w), jnp.float32),
            pltpu.SemaphoreType.DMA,
        ],
    )
    def scatter_k(msg, idx, zeros, out, idx_v, idx_cur, rows_v, acc_sh, sem):
        cid = lax.axis_index("c")
        sid = lax.axis_index("s")
        wid = cid * N_SUB + sid

        @pl.when(sid == 0)
        def _():
            pltpu.sync_copy(zeros, acc_sh)

        plsc.subcore_barrier()
        pltpu.sync_copy(idx.at[wid], idx_v)
        base = wid * EPW

        def step(i, carry):
            pltpu.sync_copy(msg.at[pl.ds(base + i * CHUNK, CHUNK)], rows_v)
            for j in range(CHUNK // 16):
                idx_cur[pl.ds(j * 16, 16)] = idx_v[i, pl.ds(j * 16, 16)]
            pltpu.sync_copy(rows_v, acc_sh.at[idx_cur], add=True)
            return carry

        lax.fori_loop(0, NCHUNK, step, 0)
        plsc.subcore_barrier()

        @pl.when(sid == 0)
        def _():
            pltpu.sync_copy(acc_sh, out.at[cid])

    return scatter_k


_EDGE_BLK = 1280
_MSG_W = 128


def _edge_body(ea_ref, xs_ref, w1_ref, b1_ref, wext_ref, rh_ref, f_ref,
               out_ref, *, with_ones):
    pid = pl.program_id(0)
    b = ea_ref.shape[0]
    row = pid * b + lax.broadcasted_iota(jnp.int32, (b, 1), 0)
    valid = (row < N_EDGES).astype(jnp.float32)
    h = jnp.maximum(
        jnp.dot(ea_ref[...], w1_ref[...], preferred_element_type=jnp.float32)
        + b1_ref[...],
        0.0,
    )
    hext = jnp.concatenate(
        [h, jnp.ones((b, 1), jnp.float32), jnp.zeros((b, 6), jnp.float32)], axis=1
    )
    y = jnp.dot(xs_ref[...], wext_ref[...], preferred_element_type=jnp.float32)
    hrep = jnp.dot(hext, rh_ref[...], preferred_element_type=jnp.float32)
    t = y * hrep
    t4 = t[:, 0:128] + t[:, 128:256] + t[:, 256:384] + t[:, 384:512]
    msg = jnp.dot(t4, f_ref[...], preferred_element_type=jnp.float32) * valid
    if with_ones:
        col = lax.broadcasted_iota(jnp.int32, (1, _MSG_W), 1)
        msg = msg + valid * (col == 16).astype(jnp.float32)
    out_ref[...] = msg


def _make_edge(c_in, with_ones):
    grid = E_PAD // _EDGE_BLK
    return pl.pallas_call(
        functools.partial(_edge_body, with_ones=with_ones),
        grid=(grid,),
        in_specs=[
            pl.BlockSpec((_EDGE_BLK, F_EDGE),
                         lambda i: (jnp.minimum(i, N_EDGES // _EDGE_BLK - 1), 0)),
            pl.BlockSpec((_EDGE_BLK, c_in), lambda i: (i, 0)),
            pl.BlockSpec((F_EDGE, 25), lambda i: (0, 0)),
            pl.BlockSpec((1, 25), lambda i: (0, 0)),
            pl.BlockSpec((c_in, 512), lambda i: (0, 0)),
            pl.BlockSpec((32, 512), lambda i: (0, 0)),
            pl.BlockSpec((128, _MSG_W), lambda i: (0, 0)),
        ],
        out_specs=pl.BlockSpec((_EDGE_BLK, _MSG_W), lambda i: (i, 0)),
        out_shape=jax.ShapeDtypeStruct((E_PAD, _MSG_W), jnp.float32),
    )


_edge1 = _make_edge(F_IN, True)
_edge23 = _make_edge(F_IN, False)


def _node1_body(p_ref, x_ref, root_ref, bias_ref, h_ref, invc_ref):
    s = p_ref[0] + p_ref[1]
    invc = 1.0 / jnp.maximum(s[:, 16:17], 1.0)
    val = (
        s[:, :16] * invc
        + jnp.dot(x_ref[...], root_ref[...], preferred_element_type=jnp.float32)
        + bias_ref[...]
    )
    h_ref[...] = jnp.concatenate(
        [jnp.maximum(val, 0.0), jnp.zeros((N_NODES, F_IN - HID), jnp.float32)],
        axis=1,
    )
    invc_ref[...] = invc


_node1 = pl.pallas_call(
    _node1_body,
    out_shape=(
        jax.ShapeDtypeStruct((N_NODES, F_IN), jnp.float32),
        jax.ShapeDtypeStruct((N_NODES, 1), jnp.float32),
    ),
)


def _node23_body(p_ref, act_ref, invc_ref, root_ref, bias_ref, out_ref, *, relu):
    s = (p_ref[0] + p_ref[1])[:, :HID]
    val = (
        s * invc_ref[...]
        + jnp.dot(act_ref[...], root_ref[...], preferred_element_type=jnp.float32)
        + bias_ref[...]
    )
    if relu:
        out_ref[...] = jnp.concatenate(
            [jnp.maximum(val, 0.0), jnp.zeros((N_NODES, F_IN - HID), jnp.float32)],
            axis=1,
        )
    else:
        out_ref[...] = val


def _make_node23(relu):
    return pl.pallas_call(
        functools.partial(_node23_body, relu=relu),
        out_shape=jax.ShapeDtypeStruct(
            (N_NODES, F_IN if relu else HID), jnp.float32
        ),
    )


_node2 = _make_node23(True)
_node3 = _make_node23(False)


def _pack_wext(w2, b2, c_in):
    w = w2.reshape(25, c_in, 16).transpose(1, 0, 2).reshape(c_in, 400)
    w = jnp.concatenate(
        [w, b2.reshape(c_in, 16), jnp.zeros((c_in, 96), jnp.float32)], axis=1
    )
    if c_in < F_IN:
        w = jnp.concatenate([w, jnp.zeros((F_IN - c_in, 512), jnp.float32)], 0)
    return w


def _selectors():
    k = jnp.arange(32)[:, None]
    j = jnp.arange(512)[None, :]
    rh = (j // 16 == k).astype(jnp.float32)
    jj = jnp.arange(128)[:, None]
    oo = jnp.arange(_MSG_W)[None, :]
    f = ((jj % 16 == oo) & (oo < 16)).astype(jnp.float32)
    return rh, f


def _pad_root(root, c_in):
    if c_in < F_IN:
        root = jnp.concatenate(
            [root, jnp.zeros((F_IN - c_in, HID), jnp.float32)], 0
        )
    return root


def kernel(x, edge_index, edge_attr,
           c1_w1, c1_b1, c1_w2, c1_b2, c1_root, c1_bias,
           c2_w1, c2_b1, c2_w2, c2_b2, c2_root, c2_bias,
           c3_w1, c3_b1, c3_w2, c3_b2, c3_root, c3_bias):
    pad = jnp.zeros((2, E_PAD - N_EDGES), jnp.int32)
    ei = jnp.concatenate([edge_index, pad], axis=1)
    src = ei[0].reshape(E_PAD // CHUNK, CHUNK)
    dst = ei[1].reshape(NW, NCHUNK, CHUNK)
    ea = edge_attr
    zeros = jnp.zeros((N_NODES, _MSG_W), jnp.float32)

    wext1 = _pack_wext(c1_w2, c1_b2, F_IN)
    wext2 = _pack_wext(c2_w2, c2_b2, HID)
    wext3 = _pack_wext(c3_w2, c3_b2, HID)
    rh, f = _selectors()

    xsrc = _make_gather(F_IN, 1)(x, src)
    msg1 = _edge1(ea, xsrc, c1_w1, c1_b1.reshape(1, 25), wext1, rh, f)
    p1 = _make_scatter(_MSG_W)(msg1, dst, zeros)
    h1, invc = _node1(p1, x, c1_root, c1_bias.reshape(1, HID))

    hs1 = _make_gather(F_IN, 1)(h1, src)
    msg2 = _edge23(ea, hs1, c2_w1, c2_b1.reshape(1, 25), wext2, rh, f)
    p2 = _make_scatter(_MSG_W)(msg2, dst, zeros)
    h2 = _node2(p2, h1, invc, _pad_root(c2_root, HID), c2_bias.reshape(1, HID))

    hs2 = _make_gather(F_IN, 1)(h2, src)
    msg3 = _edge23(ea, hs2, c3_w1, c3_b1.reshape(1, 25), wext3, rh, f)
    p3 = _make_scatter(_MSG_W)(msg3, dst, zeros)
    out = _node3(p3, h2, invc, _pad_root(c3_root, HID), c3_bias.reshape(1, HID))
    return out


# speedup vs baseline: 1.0519x; 1.0273x over previous
import functools

import jax
import jax.numpy as jnp
from jax import lax
from jax.experimental import pallas as pl
from jax.experimental.pallas import tpu as pltpu
from jax.experimental.pallas import tpu_sc as plsc

N_NODES = 10000
N_EDGES = 160000
F_IN = 128
F_EDGE = 16
HID = 16
N_CORES = 2
N_SUB = 16
NW = N_CORES * N_SUB
CHUNK = 128
NCHUNK = 40
EPW = NCHUNK * CHUNK
E_PAD = NW * EPW

@functools.lru_cache(maxsize=None)
def _mesh():
    return plsc.VectorSubcoreMesh(
        core_axis_name="c", subcore_axis_name="s",
        num_cores=N_CORES, num_subcores=N_SUB,
    )


NCHUNK_FAST = 64
NCHUNK_SLOW = 16
N_CHUNKS = E_PAD // CHUNK


@functools.lru_cache(maxsize=None)
def _make_gather(d, fast_core):
    @functools.partial(
        pl.kernel,
        out_type=jax.ShapeDtypeStruct((E_PAD, d), jnp.float32),
        mesh=_mesh(),
        scratch_types=[
            pltpu.VMEM((NCHUNK_FAST, CHUNK), jnp.int32),
            pltpu.VMEM((4, CHUNK, d), jnp.float32),
            pltpu.SemaphoreType.DMA,
            pltpu.SemaphoreType.DMA,
            pltpu.SemaphoreType.DMA,
            pltpu.SemaphoreType.DMA,
            pltpu.SemaphoreType.DMA,
            pltpu.SemaphoreType.DMA,
            pltpu.SemaphoreType.DMA,
            pltpu.SemaphoreType.DMA,
        ],
    )
    def gather_k(table, idx, out, idx_v, rows_v, *sems):
        sg = sems[:4]
        sw = sems[4:]
        cid = lax.axis_index("c")
        sid = lax.axis_index("s")
        is_fast = cid == fast_core
        my_n = jnp.where(is_fast, NCHUNK_FAST, NCHUNK_SLOW)
        base_chunk = jnp.where(
            is_fast, sid * NCHUNK_FAST,
            N_SUB * NCHUNK_FAST + sid * NCHUNK_SLOW,
        )

        @pl.when(is_fast)
        def _():
            pltpu.sync_copy(idx.at[pl.ds(sid * NCHUNK_FAST, NCHUNK_FAST)], idx_v)

        @pl.when(jnp.logical_not(is_fast))
        def _():
            pltpu.sync_copy(
                idx.at[pl.ds(N_SUB * NCHUNK_FAST + sid * NCHUNK_SLOW, NCHUNK_SLOW)],
                idx_v.at[pl.ds(0, NCHUNK_SLOW)],
            )

        for p in range(3):
            pltpu.async_copy(table.at[idx_v.at[p]], rows_v.at[p], sg[p])

        def step(g, carry):
            @pl.when(g < my_n)
            def _():
                bw = lax.rem(g + 3, 4)

                @pl.when(jnp.logical_and(g >= 1, g + 3 < my_n))
                def _():
                    for s in range(4):
                        @pl.when(bw == s)
                        def _():
                            pltpu.make_async_copy(
                                rows_v.at[s], out.at[pl.ds(0, CHUNK)], sw[s]
                            ).wait()

                @pl.when(g + 3 < my_n)
                def _():
                    for s in range(4):
                        @pl.when(bw == s)
                        def _():
                            pltpu.async_copy(
                                table.at[idx_v.at[g + 3]], rows_v.at[s], sg[s]
                            )

                bc = lax.rem(g, 4)
                for s in range(4):
                    @pl.when(bc == s)
                    def _():
                        pltpu.make_async_copy(
                            table.at[idx_v.at[g]], rows_v.at[s], sg[s]
                        ).wait()
                        pltpu.async_copy(
                            rows_v.at[s],
                            out.at[pl.ds((base_chunk + g) * CHUNK, CHUNK)],
                            sw[s],
                        )
            return carry

        lax.fori_loop(0, NCHUNK_FAST, step, 0)
        for s in range(4):
            pltpu.make_async_copy(
                rows_v.at[s], out.at[pl.ds(0, CHUNK)], sw[s]
            ).wait()

    return gather_k


@functools.lru_cache(maxsize=None)
def _make_scatter(w):
    @functools.partial(
        pl.kernel,
        out_type=jax.ShapeDtypeStruct((N_CORES, N_NODES, w), jnp.float32),
        mesh=_mesh(),
        scratch_types=[
            pltpu.VMEM((NCHUNK, CHUNK), jnp.int32),
            pltpu.VMEM((CHUNK,), jnp.int32),
            pltpu.VMEM((CHUNK, w), jnp.float32),
            pltpu.VMEM_SHARED((N_NODES, w), jnp.float32),
            pltpu.SemaphoreType.DMA,
        ],
    )
    def scatter_k(msg, idx, zeros, out, idx_v, idx_cur, rows_v, acc_sh, sem):
        cid = lax.axis_index("c")
        sid = lax.axis_index("s")
        wid = cid * N_SUB + sid

        @pl.when(sid == 0)
        def _():
            pltpu.sync_copy(zeros, acc_sh)

        plsc.subcore_barrier()
        pltpu.sync_copy(idx.at[wid], idx_v)
        base = wid * EPW

        def step(i, carry):
            pltpu.sync_copy(msg.at[pl.ds(base + i * CHUNK, CHUNK)], rows_v)
            for j in range(CHUNK // 16):
                idx_cur[pl.ds(j * 16, 16)] = idx_v[i, pl.ds(j * 16, 16)]
            pltpu.sync_copy(rows_v, acc_sh.at[idx_cur], add=True)
            return carry

        lax.fori_loop(0, NCHUNK, step, 0)
        plsc.subcore_barrier()

        @pl.when(sid == 0)
        def _():
            pltpu.sync_copy(acc_sh, out.at[cid])

    return scatter_k


_EDGE_BLK = 1280
_MSG_W = 128


def _edge_body(ea_ref, xs_ref, w1_ref, b1_ref, wext_ref, rh_ref, f_ref,
               out_ref, *, with_ones):
    pid = pl.program_id(0)
    b = ea_ref.shape[0]
    row = pid * b + lax.broadcasted_iota(jnp.int32, (b, 1), 0)
    valid = (row < N_EDGES).astype(jnp.float32)
    h = jnp.maximum(
        jnp.dot(ea_ref[...], w1_ref[...], preferred_element_type=jnp.float32)
        + b1_ref[...],
        0.0,
    )
    hext = jnp.concatenate(
        [h, jnp.ones((b, 1), jnp.float32), jnp.zeros((b, 6), jnp.float32)], axis=1
    )
    y = jnp.dot(xs_ref[...], wext_ref[...], preferred_element_type=jnp.float32)
    hrep = jnp.dot(hext, rh_ref[...], preferred_element_type=jnp.float32)
    t = y * hrep
    t4 = t[:, 0:128] + t[:, 128:256] + t[:, 256:384] + t[:, 384:512]
    msg = jnp.dot(t4, f_ref[...], preferred_element_type=jnp.float32) * valid
    if with_ones:
        col = lax.broadcasted_iota(jnp.int32, (1, _MSG_W), 1)
        msg = msg + valid * (col == 16).astype(jnp.float32)
    out_ref[...] = msg


def _make_edge(c_in, with_ones):
    grid = E_PAD // _EDGE_BLK
    return pl.pallas_call(
        functools.partial(_edge_body, with_ones=with_ones),
        grid=(grid,),
        in_specs=[
            pl.BlockSpec((_EDGE_BLK, F_EDGE),
                         lambda i: (jnp.minimum(i, N_EDGES // _EDGE_BLK - 1), 0)),
            pl.BlockSpec((_EDGE_BLK, c_in), lambda i: (i, 0)),
            pl.BlockSpec((F_EDGE, 25), lambda i: (0, 0)),
            pl.BlockSpec((1, 25), lambda i: (0, 0)),
            pl.BlockSpec((c_in, 512), lambda i: (0, 0)),
            pl.BlockSpec((32, 512), lambda i: (0, 0)),
            pl.BlockSpec((128, _MSG_W), lambda i: (0, 0)),
        ],
        out_specs=pl.BlockSpec((_EDGE_BLK, _MSG_W), lambda i: (i, 0)),
        out_shape=jax.ShapeDtypeStruct((E_PAD, _MSG_W), jnp.float32),
    )


_edge1 = _make_edge(F_IN, True)
_edge23 = _make_edge(F_IN, False)


def _node1_body(p_ref, x_ref, root_ref, bias_ref, h_ref, invc_ref):
    s = p_ref[0] + p_ref[1]
    invc = 1.0 / jnp.maximum(s[:, 16:17], 1.0)
    val = (
        s[:, :16] * invc
        + jnp.dot(x_ref[...], root_ref[...], preferred_element_type=jnp.float32)
        + bias_ref[...]
    )
    h_ref[...] = jnp.concatenate(
        [jnp.maximum(val, 0.0), jnp.zeros((N_NODES, F_IN - HID), jnp.float32)],
        axis=1,
    )
    invc_ref[...] = invc


_node1 = pl.pallas_call(
    _node1_body,
    out_shape=(
        jax.ShapeDtypeStruct((N_NODES, F_IN), jnp.float32),
        jax.ShapeDtypeStruct((N_NODES, 1), jnp.float32),
    ),
)


def _node23_body(p_ref, act_ref, invc_ref, root_ref, bias_ref, out_ref, *, relu):
    s = (p_ref[0] + p_ref[1])[:, :HID]
    val = (
        s * invc_ref[...]
        + jnp.dot(act_ref[...], root_ref[...], preferred_element_type=jnp.float32)
        + bias_ref[...]
    )
    if relu:
        out_ref[...] = jnp.concatenate(
            [jnp.maximum(val, 0.0), jnp.zeros((N_NODES, F_IN - HID), jnp.float32)],
            axis=1,
        )
    else:
        out_ref[...] = val


def _make_node23(relu):
    return pl.pallas_call(
        functools.partial(_node23_body, relu=relu),
        out_shape=jax.ShapeDtypeStruct(
            (N_NODES, F_IN if relu else HID), jnp.float32
        ),
    )


_node2 = _make_node23(True)
_node3 = _make_node23(False)


def _pack_wext(w2, b2, c_in):
    w = w2.reshape(25, c_in, 16).transpose(1, 0, 2).reshape(c_in, 400)
    w = jnp.concatenate(
        [w, b2.reshape(c_in, 16), jnp.zeros((c_in, 96), jnp.float32)], axis=1
    )
    if c_in < F_IN:
        w = jnp.concatenate([w, jnp.zeros((F_IN - c_in, 512), jnp.float32)], 0)
    return w


def _selectors():
    k = jnp.arange(32)[:, None]
    j = jnp.arange(512)[None, :]
    rh = (j // 16 == k).astype(jnp.float32)
    jj = jnp.arange(128)[:, None]
    oo = jnp.arange(_MSG_W)[None, :]
    f = ((jj % 16 == oo) & (oo < 16)).astype(jnp.float32)
    return rh, f


def _pad_root(root, c_in):
    if c_in < F_IN:
        root = jnp.concatenate(
            [root, jnp.zeros((F_IN - c_in, HID), jnp.float32)], 0
        )
    return root


def kernel(x, edge_index, edge_attr,
           c1_w1, c1_b1, c1_w2, c1_b2, c1_root, c1_bias,
           c2_w1, c2_b1, c2_w2, c2_b2, c2_root, c2_bias,
           c3_w1, c3_b1, c3_w2, c3_b2, c3_root, c3_bias):
    pad = jnp.zeros((2, E_PAD - N_EDGES), jnp.int32)
    ei = jnp.concatenate([edge_index, pad], axis=1)
    src = ei[0].reshape(E_PAD // CHUNK, CHUNK)
    dst = ei[1].reshape(NW, NCHUNK, CHUNK)
    ea = edge_attr
    zeros = jnp.zeros((N_NODES, _MSG_W), jnp.float32)

    wext1 = _pack_wext(c1_w2, c1_b2, F_IN)
    wext2 = _pack_wext(c2_w2, c2_b2, HID)
    wext3 = _pack_wext(c3_w2, c3_b2, HID)
    rh, f = _selectors()

    xsrc = _make_gather(F_IN, 1)(x, src)
    msg1 = _edge1(ea, xsrc, c1_w1, c1_b1.reshape(1, 25), wext1, rh, f)
    p1 = _make_scatter(_MSG_W)(msg1, dst, zeros)
    h1, invc = _node1(p1, x, c1_root, c1_bias.reshape(1, HID))

    hs1 = _make_gather(F_IN, 1)(h1, src)
    msg2 = _edge23(ea, hs1, c2_w1, c2_b1.reshape(1, 25), wext2, rh, f)
    p2 = _make_scatter(_MSG_W)(msg2, dst, zeros)
    h2 = _node2(p2, h1, invc, _pad_root(c2_root, HID), c2_bias.reshape(1, HID))

    hs2 = _make_gather(F_IN, 1)(h2, src)
    msg3 = _edge23(ea, hs2, c3_w1, c3_b1.reshape(1, 25), wext3, rh, f)
    p3 = _make_scatter(_MSG_W)(msg3, dst, zeros)
    out = _node3(p3, h2, invc, _pad_root(c3_root, HID), c3_bias.reshape(1, HID))
    return out


# speedup vs baseline: 1.1788x; 1.1207x over previous
import functools

import jax
import jax.numpy as jnp
from jax import lax
from jax.experimental import pallas as pl
from jax.experimental.pallas import tpu as pltpu
from jax.experimental.pallas import tpu_sc as plsc

N_NODES = 10000
N_EDGES = 160000
F_IN = 128
F_EDGE = 16
HID = 16
N_CORES = 2
N_SUB = 16
NW = N_CORES * N_SUB
CHUNK = 128
E_PAD = 163840
N_CHUNKS = E_PAD // CHUNK
HALF_CHUNKS = N_CHUNKS // 2
E_HALF = E_PAD // 2
NCHUNK_FAST = 32
NCHUNK_SLOW = 8
NCHUNK_SCAT = HALF_CHUNKS // NW
FAST_CORE = 1


@functools.lru_cache(maxsize=None)
def _mesh():
    return plsc.VectorSubcoreMesh(
        core_axis_name="c", subcore_axis_name="s",
        num_cores=N_CORES, num_subcores=N_SUB,
    )


@functools.lru_cache(maxsize=None)
def _make_gather(d, half):
    @functools.partial(
        pl.kernel,
        out_type=jax.ShapeDtypeStruct((E_HALF, d), jnp.float32),
        mesh=_mesh(),
        scratch_types=[
            pltpu.VMEM((NCHUNK_FAST, CHUNK), jnp.int32),
            pltpu.VMEM((4, CHUNK, d), jnp.float32),
        ] + [pltpu.SemaphoreType.DMA] * 8,
    )
    def gather_k(table, idx, out, idx_v, rows_v, *sems):
        sg = sems[:4]
        sw = sems[4:]
        cid = lax.axis_index("c")
        sid = lax.axis_index("s")
        is_fast = cid == FAST_CORE
        my_n = jnp.where(is_fast, NCHUNK_FAST, NCHUNK_SLOW)
        hb = half * HALF_CHUNKS
        local_base = jnp.where(
            is_fast, sid * NCHUNK_FAST,
            N_SUB * NCHUNK_FAST + sid * NCHUNK_SLOW,
        )

        @pl.when(is_fast)
        def _():
            pltpu.sync_copy(
                idx.at[pl.ds(hb + sid * NCHUNK_FAST, NCHUNK_FAST)], idx_v
            )

        @pl.when(jnp.logical_not(is_fast))
        def _():
            pltpu.sync_copy(
                idx.at[pl.ds(hb + N_SUB * NCHUNK_FAST + sid * NCHUNK_SLOW,
                             NCHUNK_SLOW)],
                idx_v.at[pl.ds(0, NCHUNK_SLOW)],
            )

        for p in range(3):
            pltpu.async_copy(table.at[idx_v.at[p]], rows_v.at[p], sg[p])

        def step(g, carry):
            @pl.when(g < my_n)
            def _():
                bw = lax.rem(g + 3, 4)

                @pl.when(jnp.logical_and(g >= 1, g + 3 < my_n))
                def _():
                    for s in range(4):
                        @pl.when(bw == s)
                        def _():
                            pltpu.make_async_copy(
                                rows_v.at[s], out.at[pl.ds(0, CHUNK)], sw[s]
                            ).wait()

                @pl.when(g + 3 < my_n)
                def _():
                    for s in range(4):
                        @pl.when(bw == s)
                        def _():
                            pltpu.async_copy(
                                table.at[idx_v.at[g + 3]], rows_v.at[s], sg[s]
                            )

                bc = lax.rem(g, 4)
                for s in range(4):
                    @pl.when(bc == s)
                    def _():
                        pltpu.make_async_copy(
                            table.at[idx_v.at[g]], rows_v.at[s], sg[s]
                        ).wait()
                        pltpu.async_copy(
                            rows_v.at[s],
                            out.at[pl.ds((local_base + g) * CHUNK, CHUNK)],
                            sw[s],
                        )
            return carry

        lax.fori_loop(0, NCHUNK_FAST, step, 0)
        for s in range(4):
            pltpu.make_async_copy(
                rows_v.at[s], out.at[pl.ds(0, CHUNK)], sw[s]
            ).wait()

    return gather_k


@functools.lru_cache(maxsize=None)
def _make_scatter(w):
    @functools.partial(
        pl.kernel,
        out_type=jax.ShapeDtypeStruct((N_CORES, N_NODES, w), jnp.float32),
        mesh=_mesh(),
        scratch_types=[
            pltpu.VMEM((NCHUNK_SCAT, CHUNK), jnp.int32),
            pltpu.VMEM((CHUNK,), jnp.int32),
            pltpu.VMEM((CHUNK, w), jnp.float32),
            pltpu.VMEM_SHARED((N_NODES, w), jnp.float32),
            pltpu.SemaphoreType.DMA,
        ],
    )
    def scatter_k(msg, idx, zeros, out, idx_v, idx_cur, rows_v, acc_sh, sem):
        cid = lax.axis_index("c")
        sid = lax.axis_index("s")
        wid = cid * N_SUB + sid

        @pl.when(sid == 0)
        def _():
            pltpu.sync_copy(zeros, acc_sh)

        plsc.subcore_barrier()
        pltpu.sync_copy(idx.at[wid], idx_v)
        base = wid * NCHUNK_SCAT

        def step(i, carry):
            pltpu.sync_copy(msg.at[pl.ds((base + i) * CHUNK, CHUNK)], rows_v)
            for j in range(CHUNK // 16):
                idx_cur[pl.ds(j * 16, 16)] = idx_v[i, pl.ds(j * 16, 16)]
            pltpu.sync_copy(rows_v, acc_sh.at[idx_cur], add=True)
            return carry

        lax.fori_loop(0, NCHUNK_SCAT, step, 0)
        plsc.subcore_barrier()

        @pl.when(sid == 0)
        def _():
            pltpu.sync_copy(acc_sh, out.at[cid])

    return scatter_k


_EDGE_BLK = 1280
_MSG_W = 128


def _edge_body(ea_ref, xs_ref, w1_ref, b1_ref, wext_ref, rh_ref, f_ref,
               out_ref, *, with_ones, half):
    pid = pl.program_id(0)
    b = ea_ref.shape[0]
    row = (half * (E_HALF // _EDGE_BLK) + pid) * b + lax.broadcasted_iota(
        jnp.int32, (b, 1), 0
    )
    valid = (row < N_EDGES).astype(jnp.float32)
    h = jnp.maximum(
        jnp.dot(ea_ref[...], w1_ref[...], preferred_element_type=jnp.float32)
        + b1_ref[...],
        0.0,
    )
    hext = jnp.concatenate(
        [h, jnp.ones((b, 1), jnp.float32), jnp.zeros((b, 6), jnp.float32)], axis=1
    )
    y = jnp.dot(xs_ref[...], wext_ref[...], preferred_element_type=jnp.float32)
    hrep = jnp.dot(hext, rh_ref[...], preferred_element_type=jnp.float32)
    t = y * hrep
    t4 = t[:, 0:128] + t[:, 128:256] + t[:, 256:384] + t[:, 384:512]
    msg = jnp.dot(t4, f_ref[...], preferred_element_type=jnp.float32) * valid
    if with_ones:
        col = lax.broadcasted_iota(jnp.int32, (1, _MSG_W), 1)
        msg = msg + valid * (col == 16).astype(jnp.float32)
    out_ref[...] = msg


@functools.lru_cache(maxsize=None)
def _make_edge(c_in, with_ones, half):
    grid = E_HALF // _EDGE_BLK
    nblk = half * grid

    def ea_map(i):
        return (jnp.minimum(nblk + i, N_EDGES // _EDGE_BLK - 1), 0)

    return pl.pallas_call(
        functools.partial(_edge_body, with_ones=with_ones, half=half),
        grid=(grid,),
        in_specs=[
            pl.BlockSpec((_EDGE_BLK, F_EDGE), ea_map),
            pl.BlockSpec((_EDGE_BLK, c_in), lambda i: (i, 0)),
            pl.BlockSpec((F_EDGE, 25), lambda i: (0, 0)),
            pl.BlockSpec((1, 25), lambda i: (0, 0)),
            pl.BlockSpec((c_in, 512), lambda i: (0, 0)),
            pl.BlockSpec((32, 512), lambda i: (0, 0)),
            pl.BlockSpec((128, _MSG_W), lambda i: (0, 0)),
        ],
        out_specs=pl.BlockSpec((_EDGE_BLK, _MSG_W), lambda i: (i, 0)),
        out_shape=jax.ShapeDtypeStruct((E_HALF, _MSG_W), jnp.float32),
    )


def _node1_body(pa_ref, pb_ref, x_ref, root_ref, bias_ref, h_ref, invc_ref):
    s = pa_ref[0] + pa_ref[1] + pb_ref[0] + pb_ref[1]
    invc = 1.0 / jnp.maximum(s[:, 16:17], 1.0)
    val = (
        s[:, :16] * invc
        + jnp.dot(x_ref[...], root_ref[...], preferred_element_type=jnp.float32)
        + bias_ref[...]
    )
    h_ref[...] = jnp.concatenate(
        [jnp.maximum(val, 0.0), jnp.zeros((N_NODES, F_IN - HID), jnp.float32)],
        axis=1,
    )
    invc_ref[...] = invc


_node1 = pl.pallas_call(
    _node1_body,
    out_shape=(
        jax.ShapeDtypeStruct((N_NODES, F_IN), jnp.float32),
        jax.ShapeDtypeStruct((N_NODES, 1), jnp.float32),
    ),
)


def _node23_body(pa_ref, pb_ref, act_ref, invc_ref, root_ref, bias_ref,
                 out_ref, *, relu):
    s = (pa_ref[0] + pa_ref[1] + pb_ref[0] + pb_ref[1])[:, :HID]
    val = (
        s * invc_ref[...]
        + jnp.dot(act_ref[...], root_ref[...], preferred_element_type=jnp.float32)
        + bias_ref[...]
    )
    if relu:
        out_ref[...] = jnp.concatenate(
            [jnp.maximum(val, 0.0), jnp.zeros((N_NODES, F_IN - HID), jnp.float32)],
            axis=1,
        )
    else:
        out_ref[...] = val


def _make_node23(relu):
    return pl.pallas_call(
        functools.partial(_node23_body, relu=relu),
        out_shape=jax.ShapeDtypeStruct(
            (N_NODES, F_IN if relu else HID), jnp.float32
        ),
    )


_node2 = _make_node23(True)
_node3 = _make_node23(False)


def _pack_wext(w2, b2, c_in):
    w = w2.reshape(25, c_in, 16).transpose(1, 0, 2).reshape(c_in, 400)
    w = jnp.concatenate(
        [w, b2.reshape(c_in, 16), jnp.zeros((c_in, 96), jnp.float32)], axis=1
    )
    if c_in < F_IN:
        w = jnp.concatenate([w, jnp.zeros((F_IN - c_in, 512), jnp.float32)], 0)
    return w


def _pad_root(root, c_in):
    if c_in < F_IN:
        root = jnp.concatenate(
            [root, jnp.zeros((F_IN - c_in, HID), jnp.float32)], 0
        )
    return root


def _selectors():
    k = jnp.arange(32)[:, None]
    j = jnp.arange(512)[None, :]
    rh = (j // 16 == k).astype(jnp.float32)
    jj = jnp.arange(128)[:, None]
    oo = jnp.arange(_MSG_W)[None, :]
    f = ((jj % 16 == oo) & (oo < 16)).astype(jnp.float32)
    return rh, f


def _layer(table, srcg, dsth, ea, w1, b1, wext, rh, f, zeros, with_ones):
    pa = []
    for hh in range(2):
        g = _make_gather(F_IN, hh)(table, srcg)
        m = _make_edge(F_IN, with_ones, hh)(ea, g, w1, b1, wext, rh, f)
        pa.append(_make_scatter(_MSG_W)(m, dsth[hh], zeros))
    return pa


def kernel(x, edge_index, edge_attr,
           c1_w1, c1_b1, c1_w2, c1_b2, c1_root, c1_bias,
           c2_w1, c2_b1, c2_w2, c2_b2, c2_root, c2_bias,
           c3_w1, c3_b1, c3_w2, c3_b2, c3_root, c3_bias):
    pad = jnp.zeros((2, E_PAD - N_EDGES), jnp.int32)
    ei = jnp.concatenate([edge_index, pad], axis=1)
    srcg = ei[0].reshape(N_CHUNKS, CHUNK)
    dsth = ei[1].reshape(2, NW, NCHUNK_SCAT, CHUNK)
    ea = edge_attr
    zeros = jnp.zeros((N_NODES, _MSG_W), jnp.float32)

    wext1 = _pack_wext(c1_w2, c1_b2, F_IN)
    wext2 = _pack_wext(c2_w2, c2_b2, HID)
    wext3 = _pack_wext(c3_w2, c3_b2, HID)
    rh, f = _selectors()

    p1a, p1b = _layer(x, srcg, dsth, ea, c1_w1, c1_b1.reshape(1, 25), wext1,
                      rh, f, zeros, True)
    h1, invc = _node1(p1a, p1b, x, c1_root, c1_bias.reshape(1, HID))

    p2a, p2b = _layer(h1, srcg, dsth, ea, c2_w1, c2_b1.reshape(1, 25), wext2,
                      rh, f, zeros, False)
    h2 = _node2(p2a, p2b, h1, invc, _pad_root(c2_root, HID),
                c2_bias.reshape(1, HID))

    p3a, p3b = _layer(h2, srcg, dsth, ea, c3_w1, c3_b1.reshape(1, 25), wext3,
                      rh, f, zeros, False)
    out = _node3(p3a, p3b, h2, invc, _pad_root(c3_root, HID),
                 c3_bias.reshape(1, HID))
    return out
